# Initial kernel scaffold; baseline (speedup 1.0000x reference)
import jax
import jax.numpy as jnp
from jax.experimental import pallas as pl


def kernel(x_s, x_t, edge_index, edge_attr, u, batch_s, W1a, b1a, W1b, b1b, W2a, b2a, W2b, b2b):
    raise NotImplementedError("write your pallas kernel here")


# speedup vs baseline: 1.7970x; 1.7970x over previous
import functools

import jax
import jax.numpy as jnp
from jax import lax
from jax.experimental import pallas as pl
from jax.experimental.pallas import tpu as pltpu
from jax.experimental.pallas import tpu_sc as plsc

N = 10000
E = 320000
H = 144
FX = 128
NW = 32
KB = 128
NROW = E // KB
RPB = NROW // NW
RPS = N // 16
NCH = H // 16

_mesh = plsc.VectorSubcoreMesh(core_axis_name="c", subcore_axis_name="s")
_sc_params = pltpu.CompilerParams(use_tc_tiling_on_sc=False)


def _leaky(x):
    return jnp.where(x >= 0, x, 0.1 * x)


def _worker_rows(wid):
    base = wid * RPB + jnp.minimum(wid, 4)
    nrow = RPB + (wid < 4).astype(jnp.int32)
    return base, nrow


def _gather_rows(y, tgt2):
    @functools.partial(
        pl.kernel,
        out_type=jax.ShapeDtypeStruct((E, FX), jnp.float32),
        mesh=_mesh,
        compiler_params=_sc_params,
        scratch_types=[
            pltpu.VMEM((KB,), jnp.int32),
            pltpu.VMEM((KB, FX), jnp.float32),
        ],
    )
    def k(y_hbm, tgt_hbm, out_hbm, idx_v, rows_v):
        wid = lax.axis_index("c") * 16 + lax.axis_index("s")
        base, nrow = _worker_rows(wid)

        def blk(i, carry):
            r = base + i
            pltpu.sync_copy(tgt_hbm.at[r], idx_v)
            pltpu.sync_copy(y_hbm.at[idx_v], rows_v)
            pltpu.sync_copy(rows_v, out_hbm.at[pl.ds(r * KB, KB)])
            return carry

        lax.fori_loop(0, nrow, blk, 0)

    return k(y, tgt2)


def _edge_mlp(g, ea, w_xt, w_ea, b1a, w1b_t, b1b):
    def body(g_ref, ea_ref, wxt_ref, wea_ref, b1a_ref, w1b_ref, b1b_ref,
             oa_ref, ob_ref):
        hp = jax.lax.Precision.HIGHEST
        a = jnp.dot(g_ref[...], wxt_ref[...], precision=hp,
                    preferred_element_type=jnp.float32) \
            + jnp.dot(ea_ref[...], wea_ref[...], precision=hp,
                      preferred_element_type=jnp.float32)
        q = _leaky(a + b1a_ref[...])
        res = jnp.dot(q, w1b_ref[...], precision=hp,
                      preferred_element_type=jnp.float32) + b1b_ref[...]
        oa_ref[...] = res[:, 0:FX]
        ob_ref[...] = res[:, 16:H]

    eb = 512
    return pl.pallas_call(
        body,
        grid=(E // eb,),
        in_specs=[
            pl.BlockSpec((eb, FX), lambda i: (i, 0)),
            pl.BlockSpec((eb, 16), lambda i: (i, 0)),
            pl.BlockSpec((FX, H), lambda i: (0, 0)),
            pl.BlockSpec((16, H), lambda i: (0, 0)),
            pl.BlockSpec((1, H), lambda i: (0, 0)),
            pl.BlockSpec((H, H), lambda i: (0, 0)),
            pl.BlockSpec((1, H), lambda i: (0, 0)),
        ],
        out_specs=[
            pl.BlockSpec((eb, FX), lambda i: (i, 0)),
            pl.BlockSpec((eb, FX), lambda i: (i, 0)),
        ],
        out_shape=[
            jax.ShapeDtypeStruct((E, FX), jnp.float32),
            jax.ShapeDtypeStruct((E, FX), jnp.float32),
        ],
    )(g, ea, w_xt, w_ea, b1a, w1b_t, b1b)


def _moments(mA, mB, src2):
    out_t = tuple(jax.ShapeDtypeStruct((N, FX), jnp.float32)
                  for _ in range(8))

    @functools.partial(
        pl.kernel,
        out_type=out_t,
        mesh=_mesh,
        compiler_params=_sc_params,
        scratch_types=[
            pltpu.VMEM((KB,), jnp.int32),
            pltpu.VMEM((KB, 16), jnp.float32),
            pltpu.VMEM((KB, 64), jnp.float32),
            pltpu.VMEM((KB, 16), jnp.float32),
            pltpu.VMEM((RPS, 64), jnp.float32),
            pltpu.VMEM_SHARED((N, 64), jnp.float32),
            pltpu.VMEM_SHARED((N, 16), jnp.float32),
        ],
    )
    def k(mA_hbm, mB_hbm, src_hbm,
          sA1, sA2, sA3, sA4, sB1, sB2, sB3, sB4,
          idx_v, m_v, pow_v, ones_v, zrow, acc, cacc):
        cid = lax.axis_index("c")
        sid = lax.axis_index("s")
        spr = NROW // 16
        rem = NROW - 16 * spr
        base = sid * spr + jnp.minimum(sid, rem)
        nrow = spr + (sid < rem).astype(jnp.int32)
        r0 = sid * RPS

        zvec = jnp.zeros((16,), jnp.float32)
        lane = lax.iota(jnp.int32, 16)
        onevec = jnp.where(lane == 0, 1.0, 0.0).astype(jnp.float32)

        def zinit(r, carry):
            for c in range(4):
                zrow[r, pl.ds(16 * c, 16)] = zvec
            return carry

        lax.fori_loop(0, RPS, zinit, 0)

        def oinit(r, carry):
            ones_v[r, pl.ds(0, 16)] = onevec
            return carry

        lax.fori_loop(0, KB, oinit, 0)

        sAs = (sA1, sA2, sA3, sA4)
        sBs = (sB1, sB2, sB3, sB4)

        for j in range(NCH):
            src_arr = mA_hbm if j < 8 else mB_hbm
            col = 16 * j if j < 8 else 112

            @pl.when(cid == (j % 2))
            def _():
                pltpu.sync_copy(zrow, acc.at[pl.ds(r0, RPS)])
                if j == 1:
                    pltpu.sync_copy(zrow.at[:, pl.ds(0, 16)],
                                    cacc.at[pl.ds(r0, RPS)])
                plsc.subcore_barrier()

                def blk(i, carry):
                    r = base + i
                    pltpu.sync_copy(src_hbm.at[r], idx_v)
                    pltpu.sync_copy(
                        src_arr.at[pl.ds(r * KB, KB), pl.ds(col, 16)], m_v)

                    def row(rr, c2):
                        m = m_v[rr, :]
                        m2 = m * m
                        pow_v[rr, pl.ds(0, 16)] = m
                        pow_v[rr, pl.ds(16, 16)] = m2
                        pow_v[rr, pl.ds(32, 16)] = m2 * m
                        pow_v[rr, pl.ds(48, 16)] = m2 * m2
                        return c2

                    lax.fori_loop(0, KB, row, 0)
                    pltpu.sync_copy(pow_v, acc.at[idx_v], add=True)
                    if j == 1:
                        pltpu.sync_copy(ones_v, cacc.at[idx_v], add=True)
                    return carry

                lax.fori_loop(0, nrow, blk, 0)
                plsc.subcore_barrier()

                for q in range(4):
                    dst = sAs[q] if j < 8 else sBs[q]
                    dcol = 16 * j if j < 8 else 0
                    pltpu.sync_copy(
                        acc.at[pl.ds(r0, RPS), pl.ds(16 * q, 16)],
                        dst.at[pl.ds(r0, RPS), pl.ds(dcol, 16)])
                if j == 1:
                    pltpu.sync_copy(cacc.at[pl.ds(r0, RPS)],
                                    sB1.at[pl.ds(r0, RPS), pl.ds(16, 16)])

    return k(mA, mB, src2)


def _node_mlp(x_s, sA, sB, bs, u,
              a_xs, a_cnt, a_m, a_sd, a_sk, a_ku, a_u, b2a, w2b_t, b2b):
    def body(xs_ref, sA1_ref, sA2_ref, sA3_ref, sA4_ref,
             sB1_ref, sB2_ref, sB3_ref, sB4_ref, bs_ref, u_ref,
             axs_ref, acnt_ref, am_ref, asd_ref, ask_ref, aku_ref, au_ref,
             b2a_ref, w2b_ref, b2b_ref, o_ref):
        s1v = jnp.concatenate([sA1_ref[...], sB1_ref[:, 0:16]], axis=1)
        s2v = jnp.concatenate([sA2_ref[...], sB2_ref[:, 0:16]], axis=1)
        s3v = jnp.concatenate([sA3_ref[...], sB3_ref[:, 0:16]], axis=1)
        s4v = jnp.concatenate([sA4_ref[...], sB4_ref[:, 0:16]], axis=1)
        count = sB1_ref[:, 16:17]
        cnt = jnp.maximum(count, 1.0)
        inv = 1.0 / cnt
        mu = s1v * inv
        m2 = s2v * inv
        mu2 = mu * mu
        var = jnp.maximum(m2 - mu2, 0.0)
        v1 = var + 1e-6
        std = jnp.sqrt(v1)
        c3m = s3v * inv - 3.0 * mu * m2 + 2.0 * mu2 * mu
        c4m = s4v * inv - 4.0 * mu * (s3v * inv) + 6.0 * mu2 * m2 \
            - 3.0 * mu2 * mu2
        skew = c3m / (std * v1)
        kurt = c4m / (v1 * v1)

        dot = lambda a, b: jnp.dot(a, b, precision=jax.lax.Precision.HIGHEST,
                                   preferred_element_type=jnp.float32)
        uproj = dot(u_ref[...], au_ref[...])
        ids = jax.lax.broadcasted_iota(jnp.int32, (bs_ref.shape[0], 64), 1)
        oneh = (bs_ref[...] == ids).astype(jnp.float32)

        h = dot(xs_ref[...], axs_ref[...]) + count * acnt_ref[...] \
            + dot(mu, am_ref[...]) + dot(std, asd_ref[...]) \
            + dot(skew, ask_ref[...]) + dot(kurt, aku_ref[...]) \
            + dot(oneh, uproj) + b2a_ref[...]
        h = _leaky(h)
        o_ref[...] = dot(h, w2b_ref[...]) + b2b_ref[...]

    nb = 1000
    full = lambda r, c: pl.BlockSpec((r, c), lambda i: (0, 0))
    row_blk = lambda c: pl.BlockSpec((nb, c), lambda i: (i, 0))
    return pl.pallas_call(
        body,
        grid=(N // nb,),
        in_specs=[
            row_blk(FX),
            row_blk(FX), row_blk(FX), row_blk(FX), row_blk(FX),
            row_blk(FX), row_blk(FX), row_blk(FX), row_blk(FX),
            pl.BlockSpec((nb, 1), lambda i: (i, 0)),
            full(64, FX),
            full(FX, FX),
            full(1, FX),
            full(H, FX),
            full(H, FX),
            full(H, FX),
            full(H, FX),
            full(FX, FX),
            full(1, FX),
            full(FX, FX),
            full(1, FX),
        ],
        out_specs=pl.BlockSpec((nb, FX), lambda i: (i, 0)),
        out_shape=jax.ShapeDtypeStruct((N, FX), jnp.float32),
    )(x_s, *sA, *sB, bs, u,
      a_xs, a_cnt, a_m, a_sd, a_sk, a_ku, a_u, b2a, w2b_t, b2b)


def kernel(x_s, x_t, edge_index, edge_attr, u, batch_s,
           W1a, b1a, W1b, b1b, W2a, b2a, W2b, b2b):
    src = edge_index[0].astype(jnp.int32).reshape(NROW, KB)
    tgt = edge_index[1].astype(jnp.int32).reshape(NROW, KB)

    g = _gather_rows(x_t, tgt)
    mA, mB = _edge_mlp(g, edge_attr, W1a[:, :FX].T, W1a[:, FX:].T,
                       b1a.reshape(1, H), W1b.T, b1b.reshape(1, H))
    outs = _moments(mA, mB, src)
    sA, sB = outs[0:4], outs[4:8]

    out = _node_mlp(
        x_s, sA, sB, batch_s.reshape(N, 1).astype(jnp.int32),
        u,
        W2a[:, 0:FX].T,
        W2a[:, FX:FX + 1].T,
        W2a[:, 129:273].T,
        W2a[:, 273:417].T,
        W2a[:, 417:561].T,
        W2a[:, 561:705].T,
        W2a[:, 705:833].T,
        b2a.reshape(1, FX),
        W2b.T,
        b2b.reshape(1, FX),
    )
    return out


# speedup vs baseline: 2.8060x; 1.5615x over previous
import functools

import jax
import jax.numpy as jnp
from jax import lax
from jax.experimental import pallas as pl
from jax.experimental.pallas import tpu as pltpu
from jax.experimental.pallas import tpu_sc as plsc

N = 10000
E = 320000
H = 144
FX = 128
NW = 32
KB = 128
NROW = E // KB
RPB = NROW // NW
RPS = N // 16
NCH = H // 16

_mesh = plsc.VectorSubcoreMesh(core_axis_name="c", subcore_axis_name="s")
_sc_params = pltpu.CompilerParams(use_tc_tiling_on_sc=False)


def _leaky(x):
    return jnp.where(x >= 0, x, 0.1 * x)


def _worker_rows(wid):
    base = wid * RPB + jnp.minimum(wid, 4)
    nrow = RPB + (wid < 4).astype(jnp.int32)
    return base, nrow


def _gather_rows(y, tgt2):
    @functools.partial(
        pl.kernel,
        out_type=jax.ShapeDtypeStruct((E, FX), jnp.float32),
        mesh=_mesh,
        compiler_params=_sc_params,
        scratch_types=[
            pltpu.VMEM((RPB + 1, KB), jnp.int32),
            [pltpu.VMEM((KB, FX), jnp.float32) for _ in range(3)],
            pltpu.SemaphoreType.DMA((3,)),
            pltpu.SemaphoreType.DMA((3,)),
        ],
    )
    def k(y_hbm, tgt_hbm, out_hbm, idx_all, rows, g_sem, w_sem):
        wid = lax.axis_index("c") * 16 + lax.axis_index("s")
        base, _ = _worker_rows(wid)
        pltpu.sync_copy(tgt_hbm.at[pl.ds(base, RPB)],
                        idx_all.at[pl.ds(0, RPB)])

        @pl.when(wid < 4)
        def _():
            pltpu.sync_copy(tgt_hbm.at[base + RPB], idx_all.at[RPB])

        def slot(i3, carry):
            for bo in range(3):
                i = 3 * i3 + bo
                b = bo
                bj = (bo + 2) % 3

                @pl.when(i < RPB)
                def _():
                    @pl.when(i >= 3)
                    def _():
                        pltpu.make_async_copy(
                            rows[b],
                            out_hbm.at[pl.ds((base + i - 3) * KB, KB)],
                            w_sem.at[b]).wait()
                    pltpu.async_copy(y_hbm.at[idx_all.at[i]], rows[b],
                                     g_sem.at[b])

                @pl.when((i >= 1) & (i <= RPB))
                def _():
                    j = i - 1
                    pltpu.make_async_copy(y_hbm.at[idx_all.at[j]], rows[bj],
                                          g_sem.at[bj]).wait()
                    pltpu.async_copy(rows[bj],
                                     out_hbm.at[pl.ds((base + j) * KB, KB)],
                                     w_sem.at[bj])
            return carry

        lax.fori_loop(0, (RPB + 1 + 2) // 3, slot, 0)
        for kk in (RPB - 3, RPB - 2, RPB - 1):
            b = kk % 3
            pltpu.make_async_copy(
                rows[b], out_hbm.at[pl.ds((base + kk) * KB, KB)],
                w_sem.at[b]).wait()

        @pl.when(wid < 4)
        def _():
            pltpu.sync_copy(y_hbm.at[idx_all.at[RPB]], rows[0])
            pltpu.sync_copy(rows[0], out_hbm.at[pl.ds((base + RPB) * KB, KB)])

    return k(y, tgt2)


def _edge_mlp(g, ea, w_xt, w_ea, b1a, w1b_t, b1b):
    def body(g_ref, ea_ref, wxt_ref, wea_ref, b1a_ref, w1b_ref, b1b_ref,
             oa_ref, ob_ref):
        hp = jax.lax.Precision.HIGHEST
        a = jnp.dot(g_ref[...], wxt_ref[...], precision=hp,
                    preferred_element_type=jnp.float32) \
            + jnp.dot(ea_ref[...], wea_ref[...], precision=hp,
                      preferred_element_type=jnp.float32)
        q = _leaky(a + b1a_ref[...])
        res = jnp.dot(q, w1b_ref[...], precision=hp,
                      preferred_element_type=jnp.float32) + b1b_ref[...]
        oa_ref[...] = res[:, 0:FX]
        ob_ref[...] = res[:, 16:H]

    eb = 512
    return pl.pallas_call(
        body,
        grid=(E // eb,),
        in_specs=[
            pl.BlockSpec((eb, FX), lambda i: (i, 0)),
            pl.BlockSpec((eb, 16), lambda i: (i, 0)),
            pl.BlockSpec((FX, H), lambda i: (0, 0)),
            pl.BlockSpec((16, H), lambda i: (0, 0)),
            pl.BlockSpec((1, H), lambda i: (0, 0)),
            pl.BlockSpec((H, H), lambda i: (0, 0)),
            pl.BlockSpec((1, H), lambda i: (0, 0)),
        ],
        out_specs=[
            pl.BlockSpec((eb, FX), lambda i: (i, 0)),
            pl.BlockSpec((eb, FX), lambda i: (i, 0)),
        ],
        out_shape=[
            jax.ShapeDtypeStruct((E, FX), jnp.float32),
            jax.ShapeDtypeStruct((E, FX), jnp.float32),
        ],
    )(g, ea, w_xt, w_ea, b1a, w1b_t, b1b)


def _moments(mA, mB, src2):
    out_t = tuple(jax.ShapeDtypeStruct((N, FX), jnp.float32)
                  for _ in range(8))

    SPR = NROW // 16
    REM = NROW - 16 * SPR
    HR = NROW // 2
    SPR8 = HR // 16
    REM8 = HR - 16 * SPR8

    @functools.partial(
        pl.kernel,
        out_type=out_t,
        mesh=_mesh,
        compiler_params=_sc_params,
        scratch_types=[
            pltpu.VMEM((SPR + 1, KB), jnp.int32),
            [pltpu.VMEM((KB, 16), jnp.float32) for _ in range(3)],
            [pltpu.VMEM((KB, 64), jnp.float32) for _ in range(3)],
            pltpu.VMEM((KB, 16), jnp.float32),
            pltpu.VMEM((RPS // 5, 64), jnp.float32),
            pltpu.VMEM_SHARED((N, 64), jnp.float32),
            pltpu.VMEM_SHARED((N, 16), jnp.float32),
            pltpu.SemaphoreType.DMA((3,)),
            pltpu.SemaphoreType.DMA((3,)),
            pltpu.SemaphoreType.DMA((3,)),
        ],
    )
    def k(mA_hbm, mB_hbm, src_hbm,
          sA1, sA2, sA3, sA4, sB1, sB2, sB3, sB4,
          idx_all, m_v, pow_v, ones_v, zrow, acc, cacc,
          m_sem, sc_sem, cnt_sem):
        cid = lax.axis_index("c")
        sid = lax.axis_index("s")
        r0 = sid * RPS

        base_f = sid * SPR + jnp.minimum(sid, REM)
        base_8 = cid * HR + sid * SPR8 + jnp.minimum(sid, REM8)

        pltpu.sync_copy(src_hbm.at[pl.ds(base_f, SPR)],
                        idx_all.at[pl.ds(0, SPR)])

        @pl.when(sid < REM)
        def _():
            pltpu.sync_copy(src_hbm.at[base_f + SPR], idx_all.at[SPR])

        zvec = jnp.zeros((16,), jnp.float32)
        lane = lax.iota(jnp.int32, 16)
        onevec = jnp.where(lane == 0, 1.0, 0.0).astype(jnp.float32)

        def zinit(r, carry):
            for c in range(4):
                zrow[r, pl.ds(16 * c, 16)] = zvec
            return carry

        lax.fori_loop(0, RPS // 5, zinit, 0)

        def zero_acc():
            for t in range(5):
                pltpu.sync_copy(zrow,
                                acc.at[pl.ds(r0 + t * (RPS // 5), RPS // 5)])

        def zero_cacc():
            for t in range(5):
                pltpu.sync_copy(zrow.at[:, pl.ds(0, 16)],
                                cacc.at[pl.ds(r0 + t * (RPS // 5), RPS // 5)])

        def oinit(r, carry):
            ones_v[r, pl.ds(0, 16)] = onevec
            return carry

        lax.fori_loop(0, KB, oinit, 0)

        sAs = (sA1, sA2, sA3, sA4)
        sBs = (sB1, sB2, sB3, sB4)

        def compute_pows(b):
            def row(rr, c2):
                m = m_v[b][rr, :]
                m2 = m * m
                pow_v[b][rr, pl.ds(0, 16)] = m
                pow_v[b][rr, pl.ds(16, 16)] = m2
                pow_v[b][rr, pl.ds(32, 16)] = m2 * m
                pow_v[b][rr, pl.ds(48, 16)] = m2 * m2
                return c2

            lax.fori_loop(0, KB, row, 0, unroll=8)

        def run_chunk(src_arr, col, base, nblk, docount):
            def mload_copy(i, b):
                r = base + i
                return pltpu.make_async_copy(
                    src_arr.at[pl.ds(r * KB, KB), pl.ds(col, 16)],
                    m_v[b], m_sem.at[b])

            def scat_copy(i, b):
                return pltpu.make_async_copy(
                    pow_v[b], acc.at[idx_all.at[i]], sc_sem.at[b])

            def slot(i3, carry):
                for bo in range(3):
                    i = 3 * i3 + bo
                    b = bo
                    bj = (bo + 2) % 3

                    @pl.when(i < nblk)
                    def _():
                        mload_copy(i, b).start()

                    @pl.when((i >= 1) & (i <= nblk))
                    def _():
                        j = i - 1
                        mload_copy(j, bj).wait()

                        @pl.when(j >= 3)
                        def _():
                            scat_copy(j - 3, bj).wait()
                            if docount:
                                pltpu.make_async_copy(
                                    ones_v, cacc.at[idx_all.at[j - 3]],
                                    cnt_sem.at[bj]).wait()

                        compute_pows(bj)
                        pltpu.async_copy(pow_v[bj], acc.at[idx_all.at[j]],
                                         sc_sem.at[bj], add=True)
                        if docount:
                            pltpu.async_copy(ones_v, cacc.at[idx_all.at[j]],
                                             cnt_sem.at[bj], add=True)
                return carry

            lax.fori_loop(0, (nblk + 1 + 2) // 3, slot, 0)
            for kk in (nblk - 3, nblk - 2, nblk - 1):
                b = kk % 3
                scat_copy(kk, b).wait()
                if docount:
                    pltpu.make_async_copy(ones_v, cacc.at[idx_all.at[kk]],
                                          cnt_sem.at[b]).wait()

        def tail_block(src_arr, col, r, irow, docount):
            pltpu.sync_copy(src_arr.at[pl.ds(r * KB, KB), pl.ds(col, 16)],
                            m_v[0])
            compute_pows(0)
            pltpu.sync_copy(pow_v[0], acc.at[idx_all.at[irow]], add=True)
            if docount:
                pltpu.sync_copy(ones_v, cacc.at[idx_all.at[irow]], add=True)

        for j in range(NCH):
            src_arr = mA_hbm if j < 8 else mB_hbm
            col = 16 * j if j < 8 else 112

            if j < 8:
                @pl.when(cid == (j % 2))
                def _():
                    zero_acc()
                    if j == 1:
                        zero_cacc()
                    plsc.subcore_barrier()
                    run_chunk(src_arr, col, base_f, SPR, j == 1)

                    @pl.when(sid < REM)
                    def _():
                        tail_block(src_arr, col, base_f + SPR, SPR, j == 1)

                    plsc.subcore_barrier()
                    for q in range(4):
                        pltpu.sync_copy(
                            acc.at[pl.ds(r0, RPS), pl.ds(16 * q, 16)],
                            sAs[q].at[pl.ds(r0, RPS), pl.ds(16 * j, 16)])
                    if j == 1:
                        pltpu.sync_copy(cacc.at[pl.ds(r0, RPS)],
                                        sB1.at[pl.ds(r0, RPS),
                                               pl.ds(16, 16)])
            else:
                pltpu.sync_copy(src_hbm.at[pl.ds(base_8, SPR8)],
                                idx_all.at[pl.ds(0, SPR8)])

                @pl.when(sid < REM8)
                def _():
                    pltpu.sync_copy(src_hbm.at[base_8 + SPR8],
                                    idx_all.at[SPR8])

                zero_acc()
                plsc.subcore_barrier()
                run_chunk(src_arr, col, base_8, SPR8, False)

                @pl.when(sid < REM8)
                def _():
                    tail_block(src_arr, col, base_8 + SPR8, SPR8, False)

                plsc.subcore_barrier()
                dcol = cid * 32
                for q in range(4):
                    pltpu.sync_copy(
                        acc.at[pl.ds(r0, RPS), pl.ds(16 * q, 16)],
                        sBs[q].at[pl.ds(r0, RPS), pl.ds(dcol, 16)])

    return k(mA, mB, src2)


def _node_mlp(x_s, sA, sB, bs, u,
              a_xs, a_cnt, a_m, a_sd, a_sk, a_ku, a_u, b2a, w2b_t, b2b):
    def body(xs_ref, sA1_ref, sA2_ref, sA3_ref, sA4_ref,
             sB1_ref, sB2_ref, sB3_ref, sB4_ref, bs_ref, u_ref,
             axs_ref, acnt_ref, am_ref, asd_ref, ask_ref, aku_ref, au_ref,
             b2a_ref, w2b_ref, b2b_ref, o_ref):
        tail = lambda b_ref: b_ref[:, 0:16] + b_ref[:, 32:48]
        s1v = jnp.concatenate([sA1_ref[...], tail(sB1_ref)], axis=1)
        s2v = jnp.concatenate([sA2_ref[...], tail(sB2_ref)], axis=1)
        s3v = jnp.concatenate([sA3_ref[...], tail(sB3_ref)], axis=1)
        s4v = jnp.concatenate([sA4_ref[...], tail(sB4_ref)], axis=1)
        count = sB1_ref[:, 16:17]
        cnt = jnp.maximum(count, 1.0)
        inv = 1.0 / cnt
        mu = s1v * inv
        m2 = s2v * inv
        mu2 = mu * mu
        var = jnp.maximum(m2 - mu2, 0.0)
        v1 = var + 1e-6
        std = jnp.sqrt(v1)
        c3m = s3v * inv - 3.0 * mu * m2 + 2.0 * mu2 * mu
        c4m = s4v * inv - 4.0 * mu * (s3v * inv) + 6.0 * mu2 * m2 \
            - 3.0 * mu2 * mu2
        skew = c3m / (std * v1)
        kurt = c4m / (v1 * v1)

        dot = lambda a, b: jnp.dot(a, b, precision=jax.lax.Precision.HIGHEST,
                                   preferred_element_type=jnp.float32)
        uproj = dot(u_ref[...], au_ref[...])
        ids = jax.lax.broadcasted_iota(jnp.int32, (bs_ref.shape[0], 64), 1)
        oneh = (bs_ref[...] == ids).astype(jnp.float32)

        h = dot(xs_ref[...], axs_ref[...]) + count * acnt_ref[...] \
            + dot(mu, am_ref[...]) + dot(std, asd_ref[...]) \
            + dot(skew, ask_ref[...]) + dot(kurt, aku_ref[...]) \
            + dot(oneh, uproj) + b2a_ref[...]
        h = _leaky(h)
        o_ref[...] = dot(h, w2b_ref[...]) + b2b_ref[...]

    nb = 1000
    full = lambda r, c: pl.BlockSpec((r, c), lambda i: (0, 0))
    row_blk = lambda c: pl.BlockSpec((nb, c), lambda i: (i, 0))
    return pl.pallas_call(
        body,
        grid=(N // nb,),
        in_specs=[
            row_blk(FX),
            row_blk(FX), row_blk(FX), row_blk(FX), row_blk(FX),
            row_blk(FX), row_blk(FX), row_blk(FX), row_blk(FX),
            pl.BlockSpec((nb, 1), lambda i: (i, 0)),
            full(64, FX),
            full(FX, FX),
            full(1, FX),
            full(H, FX),
            full(H, FX),
            full(H, FX),
            full(H, FX),
            full(FX, FX),
            full(1, FX),
            full(FX, FX),
            full(1, FX),
        ],
        out_specs=pl.BlockSpec((nb, FX), lambda i: (i, 0)),
        out_shape=jax.ShapeDtypeStruct((N, FX), jnp.float32),
    )(x_s, *sA, *sB, bs, u,
      a_xs, a_cnt, a_m, a_sd, a_sk, a_ku, a_u, b2a, w2b_t, b2b)


def kernel(x_s, x_t, edge_index, edge_attr, u, batch_s,
           W1a, b1a, W1b, b1b, W2a, b2a, W2b, b2b):
    src = edge_index[0].astype(jnp.int32).reshape(NROW, KB)
    tgt = edge_index[1].astype(jnp.int32).reshape(NROW, KB)

    g = _gather_rows(x_t, tgt)
    mA, mB = _edge_mlp(g, edge_attr, W1a[:, :FX].T, W1a[:, FX:].T,
                       b1a.reshape(1, H), W1b.T, b1b.reshape(1, H))
    outs = _moments(mA, mB, src)
    sA, sB = outs[0:4], outs[4:8]

    out = _node_mlp(
        x_s, sA, sB, batch_s.reshape(N, 1).astype(jnp.int32),
        u,
        W2a[:, 0:FX].T,
        W2a[:, FX:FX + 1].T,
        W2a[:, 129:273].T,
        W2a[:, 273:417].T,
        W2a[:, 417:561].T,
        W2a[:, 561:705].T,
        W2a[:, 705:833].T,
        b2a.reshape(1, FX),
        W2b.T,
        b2b.reshape(1, FX),
    )
    return out


# speedup vs baseline: 3.3679x; 1.2002x over previous
import functools

import jax
import jax.numpy as jnp
from jax import lax
from jax.experimental import pallas as pl
from jax.experimental.pallas import tpu as pltpu
from jax.experimental.pallas import tpu_sc as plsc

N = 10000
E = 320000
H = 144
FX = 128
NW = 32
KB = 128
NROW = E // KB
RPB = NROW // NW
RPS = N // 16
NCH = H // 16

_mesh = plsc.VectorSubcoreMesh(core_axis_name="c", subcore_axis_name="s")
_sc_params = pltpu.CompilerParams(use_tc_tiling_on_sc=False)


def _leaky(x):
    return jnp.where(x >= 0, x, 0.1 * x)


def _worker_rows(wid):
    base = wid * RPB + jnp.minimum(wid, 4)
    nrow = RPB + (wid < 4).astype(jnp.int32)
    return base, nrow


def _gather_rows(y, tgt2):
    @functools.partial(
        pl.kernel,
        out_type=jax.ShapeDtypeStruct((E, FX), jnp.float32),
        mesh=_mesh,
        compiler_params=_sc_params,
        scratch_types=[
            pltpu.VMEM((RPB + 1, KB), jnp.int32),
            [pltpu.VMEM((KB, FX), jnp.float32) for _ in range(3)],
            pltpu.SemaphoreType.DMA((3,)),
            pltpu.SemaphoreType.DMA((3,)),
        ],
    )
    def k(y_hbm, tgt_hbm, out_hbm, idx_all, rows, g_sem, w_sem):
        wid = lax.axis_index("c") * 16 + lax.axis_index("s")
        base, _ = _worker_rows(wid)
        pltpu.sync_copy(tgt_hbm.at[pl.ds(base, RPB)],
                        idx_all.at[pl.ds(0, RPB)])

        @pl.when(wid < 4)
        def _():
            pltpu.sync_copy(tgt_hbm.at[base + RPB], idx_all.at[RPB])

        def slot(i3, carry):
            for bo in range(3):
                i = 3 * i3 + bo
                b = bo
                bj = (bo + 2) % 3

                @pl.when(i < RPB)
                def _():
                    @pl.when(i >= 3)
                    def _():
                        pltpu.make_async_copy(
                            rows[b],
                            out_hbm.at[pl.ds((base + i - 3) * KB, KB)],
                            w_sem.at[b]).wait()
                    pltpu.async_copy(y_hbm.at[idx_all.at[i]], rows[b],
                                     g_sem.at[b])

                @pl.when((i >= 1) & (i <= RPB))
                def _():
                    j = i - 1
                    pltpu.make_async_copy(y_hbm.at[idx_all.at[j]], rows[bj],
                                          g_sem.at[bj]).wait()
                    pltpu.async_copy(rows[bj],
                                     out_hbm.at[pl.ds((base + j) * KB, KB)],
                                     w_sem.at[bj])
            return carry

        lax.fori_loop(0, (RPB + 1 + 2) // 3, slot, 0)
        for kk in (RPB - 3, RPB - 2, RPB - 1):
            b = kk % 3
            pltpu.make_async_copy(
                rows[b], out_hbm.at[pl.ds((base + kk) * KB, KB)],
                w_sem.at[b]).wait()

        @pl.when(wid < 4)
        def _():
            pltpu.sync_copy(y_hbm.at[idx_all.at[RPB]], rows[0])
            pltpu.sync_copy(rows[0], out_hbm.at[pl.ds((base + RPB) * KB, KB)])

    return k(y, tgt2)


def _edge_mlp(g, ea, w1a_t, b1a, w1b_t, b1b):
    def body(g_ref, ea_ref, w1a_ref, b1a_ref, w1b_ref, b1b_ref,
             oa_ref, ob_ref):
        hp = jax.lax.Precision.HIGHEST
        x = jnp.concatenate([g_ref[...], ea_ref[...]], axis=1)
        a = jnp.dot(x, w1a_ref[...], precision=hp,
                    preferred_element_type=jnp.float32)
        q = _leaky(a + b1a_ref[...])
        res = jnp.dot(q, w1b_ref[...], precision=hp,
                      preferred_element_type=jnp.float32) + b1b_ref[...]
        oa_ref[...] = res[:, 0:FX]
        ob_ref[...] = res[:, 16:H]

    eb = 1600
    return pl.pallas_call(
        body,
        grid=(E // eb,),
        in_specs=[
            pl.BlockSpec((eb, FX), lambda i: (i, 0)),
            pl.BlockSpec((eb, 16), lambda i: (i, 0)),
            pl.BlockSpec((H, H), lambda i: (0, 0)),
            pl.BlockSpec((1, H), lambda i: (0, 0)),
            pl.BlockSpec((H, H), lambda i: (0, 0)),
            pl.BlockSpec((1, H), lambda i: (0, 0)),
        ],
        out_specs=[
            pl.BlockSpec((eb, FX), lambda i: (i, 0)),
            pl.BlockSpec((eb, FX), lambda i: (i, 0)),
        ],
        out_shape=[
            jax.ShapeDtypeStruct((E, FX), jnp.float32),
            jax.ShapeDtypeStruct((E, FX), jnp.float32),
        ],
    )(g, ea, w1a_t, b1a, w1b_t, b1b)


def _moments(mA, mB, src2):
    out_t = tuple(jax.ShapeDtypeStruct((N, FX), jnp.float32)
                  for _ in range(8))

    SPR = NROW // 16
    REM = NROW - 16 * SPR
    HR = NROW // 2
    SPR8 = HR // 16
    REM8 = HR - 16 * SPR8

    @functools.partial(
        pl.kernel,
        out_type=out_t,
        mesh=_mesh,
        compiler_params=_sc_params,
        scratch_types=[
            pltpu.VMEM((SPR + 1, KB), jnp.int32),
            [pltpu.VMEM((KB, 16), jnp.float32) for _ in range(3)],
            [pltpu.VMEM((KB, 64), jnp.float32) for _ in range(3)],
            pltpu.VMEM((KB, 16), jnp.float32),
            pltpu.VMEM((RPS // 5, 64), jnp.float32),
            pltpu.VMEM_SHARED((N, 64), jnp.float32),
            pltpu.VMEM_SHARED((N, 16), jnp.float32),
            pltpu.SemaphoreType.DMA((3,)),
            pltpu.SemaphoreType.DMA((3,)),
            pltpu.SemaphoreType.DMA((3,)),
        ],
    )
    def k(mA_hbm, mB_hbm, src_hbm,
          sA1, sA2, sA3, sA4, sB1, sB2, sB3, sB4,
          idx_all, m_v, pow_v, ones_v, zrow, acc, cacc,
          m_sem, sc_sem, cnt_sem):
        cid = lax.axis_index("c")
        sid = lax.axis_index("s")
        r0 = sid * RPS

        base_f = sid * SPR + jnp.minimum(sid, REM)
        base_8 = cid * HR + sid * SPR8 + jnp.minimum(sid, REM8)

        pltpu.sync_copy(src_hbm.at[pl.ds(base_f, SPR)],
                        idx_all.at[pl.ds(0, SPR)])

        @pl.when(sid < REM)
        def _():
            pltpu.sync_copy(src_hbm.at[base_f + SPR], idx_all.at[SPR])

        zvec = jnp.zeros((16,), jnp.float32)
        lane = lax.iota(jnp.int32, 16)
        onevec = jnp.where(lane == 0, 1.0, 0.0).astype(jnp.float32)

        def zinit(r, carry):
            for c in range(4):
                zrow[r, pl.ds(16 * c, 16)] = zvec
            return carry

        lax.fori_loop(0, RPS // 5, zinit, 0)

        def zero_acc():
            for t in range(5):
                pltpu.sync_copy(zrow,
                                acc.at[pl.ds(r0 + t * (RPS // 5), RPS // 5)])

        def zero_cacc():
            for t in range(5):
                pltpu.sync_copy(zrow.at[:, pl.ds(0, 16)],
                                cacc.at[pl.ds(r0 + t * (RPS // 5), RPS // 5)])

        def oinit(r, carry):
            ones_v[r, pl.ds(0, 16)] = onevec
            return carry

        lax.fori_loop(0, KB, oinit, 0)

        sAs = (sA1, sA2, sA3, sA4)
        sBs = (sB1, sB2, sB3, sB4)

        def compute_pows(b):
            def row(rr, c2):
                m = m_v[b][rr, :]
                m2 = m * m
                pow_v[b][rr, pl.ds(0, 16)] = m
                pow_v[b][rr, pl.ds(16, 16)] = m2
                pow_v[b][rr, pl.ds(32, 16)] = m2 * m
                pow_v[b][rr, pl.ds(48, 16)] = m2 * m2
                return c2

            lax.fori_loop(0, KB, row, 0, unroll=8)

        def run_chunk(src_arr, col, base, nblk, docount):
            def mload_copy(i, b):
                r = base + i
                return pltpu.make_async_copy(
                    src_arr.at[pl.ds(r * KB, KB), pl.ds(col, 16)],
                    m_v[b], m_sem.at[b])

            def scat_copy(i, b):
                return pltpu.make_async_copy(
                    pow_v[b], acc.at[idx_all.at[i]], sc_sem.at[b])

            def slot(i3, carry):
                for bo in range(3):
                    i = 3 * i3 + bo
                    b = bo
                    bj = (bo + 2) % 3

                    @pl.when(i < nblk)
                    def _():
                        mload_copy(i, b).start()

                    @pl.when((i >= 1) & (i <= nblk))
                    def _():
                        j = i - 1
                        mload_copy(j, bj).wait()

                        @pl.when(j >= 3)
                        def _():
                            scat_copy(j - 3, bj).wait()
                            if docount:
                                pltpu.make_async_copy(
                                    ones_v, cacc.at[idx_all.at[j - 3]],
                                    cnt_sem.at[bj]).wait()

                        compute_pows(bj)
                        pltpu.async_copy(pow_v[bj], acc.at[idx_all.at[j]],
                                         sc_sem.at[bj], add=True)
                        if docount:
                            pltpu.async_copy(ones_v, cacc.at[idx_all.at[j]],
                                             cnt_sem.at[bj], add=True)
                return carry

            lax.fori_loop(0, (nblk + 1 + 2) // 3, slot, 0)
            for kk in (nblk - 3, nblk - 2, nblk - 1):
                b = kk % 3
                scat_copy(kk, b).wait()
                if docount:
                    pltpu.make_async_copy(ones_v, cacc.at[idx_all.at[kk]],
                                          cnt_sem.at[b]).wait()

        def tail_block(src_arr, col, r, irow, docount):
            pltpu.sync_copy(src_arr.at[pl.ds(r * KB, KB), pl.ds(col, 16)],
                            m_v[0])
            compute_pows(0)
            pltpu.sync_copy(pow_v[0], acc.at[idx_all.at[irow]], add=True)
            if docount:
                pltpu.sync_copy(ones_v, cacc.at[idx_all.at[irow]], add=True)

        for j in range(NCH):
            src_arr = mA_hbm if j < 8 else mB_hbm
            col = 16 * j if j < 8 else 112

            if j < 8:
                @pl.when(cid == (j % 2))
                def _():
                    zero_acc()
                    if j == 1:
                        zero_cacc()
                    plsc.subcore_barrier()
                    run_chunk(src_arr, col, base_f, SPR, j == 1)

                    @pl.when(sid < REM)
                    def _():
                        tail_block(src_arr, col, base_f + SPR, SPR, j == 1)

                    plsc.subcore_barrier()
                    for q in range(4):
                        pltpu.sync_copy(
                            acc.at[pl.ds(r0, RPS), pl.ds(16 * q, 16)],
                            sAs[q].at[pl.ds(r0, RPS), pl.ds(16 * j, 16)])
                    if j == 1:
                        pltpu.sync_copy(cacc.at[pl.ds(r0, RPS)],
                                        sB1.at[pl.ds(r0, RPS),
                                               pl.ds(16, 16)])
            else:
                pltpu.sync_copy(src_hbm.at[pl.ds(base_8, SPR8)],
                                idx_all.at[pl.ds(0, SPR8)])

                @pl.when(sid < REM8)
                def _():
                    pltpu.sync_copy(src_hbm.at[base_8 + SPR8],
                                    idx_all.at[SPR8])

                zero_acc()
                plsc.subcore_barrier()
                run_chunk(src_arr, col, base_8, SPR8, False)

                @pl.when(sid < REM8)
                def _():
                    tail_block(src_arr, col, base_8 + SPR8, SPR8, False)

                plsc.subcore_barrier()
                dcol = cid * 32
                for q in range(4):
                    pltpu.sync_copy(
                        acc.at[pl.ds(r0, RPS), pl.ds(16 * q, 16)],
                        sBs[q].at[pl.ds(r0, RPS), pl.ds(dcol, 16)])

    return k(mA, mB, src2)


def _node_mlp(x_s, sA, sB, bs, u,
              a_xs, a_cnt, a_m, a_sd, a_sk, a_ku, a_u, b2a, w2b_t, b2b):
    def body(xs_ref, sA1_ref, sA2_ref, sA3_ref, sA4_ref,
             sB1_ref, sB2_ref, sB3_ref, sB4_ref, bs_ref, u_ref,
             axs_ref, acnt_ref, am_ref, asd_ref, ask_ref, aku_ref, au_ref,
             b2a_ref, w2b_ref, b2b_ref, o_ref):
        tail = lambda b_ref: b_ref[:, 0:16] + b_ref[:, 32:48]
        s1v = jnp.concatenate([sA1_ref[...], tail(sB1_ref)], axis=1)
        s2v = jnp.concatenate([sA2_ref[...], tail(sB2_ref)], axis=1)
        s3v = jnp.concatenate([sA3_ref[...], tail(sB3_ref)], axis=1)
        s4v = jnp.concatenate([sA4_ref[...], tail(sB4_ref)], axis=1)
        count = sB1_ref[:, 16:17]
        cnt = jnp.maximum(count, 1.0)
        inv = 1.0 / cnt
        mu = s1v * inv
        m2 = s2v * inv
        mu2 = mu * mu
        var = jnp.maximum(m2 - mu2, 0.0)
        v1 = var + 1e-6
        std = jnp.sqrt(v1)
        c3m = s3v * inv - 3.0 * mu * m2 + 2.0 * mu2 * mu
        c4m = s4v * inv - 4.0 * mu * (s3v * inv) + 6.0 * mu2 * m2 \
            - 3.0 * mu2 * mu2
        skew = c3m / (std * v1)
        kurt = c4m / (v1 * v1)

        dot = lambda a, b: jnp.dot(a, b, precision=jax.lax.Precision.HIGHEST,
                                   preferred_element_type=jnp.float32)
        uproj = dot(u_ref[...], au_ref[...])
        ids = jax.lax.broadcasted_iota(jnp.int32, (bs_ref.shape[0], 64), 1)
        oneh = (bs_ref[...] == ids).astype(jnp.float32)

        h = dot(xs_ref[...], axs_ref[...]) + count * acnt_ref[...] \
            + dot(mu, am_ref[...]) + dot(std, asd_ref[...]) \
            + dot(skew, ask_ref[...]) + dot(kurt, aku_ref[...]) \
            + dot(oneh, uproj) + b2a_ref[...]
        h = _leaky(h)
        o_ref[...] = dot(h, w2b_ref[...]) + b2b_ref[...]

    nb = 1000
    full = lambda r, c: pl.BlockSpec((r, c), lambda i: (0, 0))
    row_blk = lambda c: pl.BlockSpec((nb, c), lambda i: (i, 0))
    return pl.pallas_call(
        body,
        grid=(N // nb,),
        in_specs=[
            row_blk(FX),
            row_blk(FX), row_blk(FX), row_blk(FX), row_blk(FX),
            row_blk(FX), row_blk(FX), row_blk(FX), row_blk(FX),
            pl.BlockSpec((nb, 1), lambda i: (i, 0)),
            full(64, FX),
            full(FX, FX),
            full(1, FX),
            full(H, FX),
            full(H, FX),
            full(H, FX),
            full(H, FX),
            full(FX, FX),
            full(1, FX),
            full(FX, FX),
            full(1, FX),
        ],
        out_specs=pl.BlockSpec((nb, FX), lambda i: (i, 0)),
        out_shape=jax.ShapeDtypeStruct((N, FX), jnp.float32),
    )(x_s, *sA, *sB, bs, u,
      a_xs, a_cnt, a_m, a_sd, a_sk, a_ku, a_u, b2a, w2b_t, b2b)


def kernel(x_s, x_t, edge_index, edge_attr, u, batch_s,
           W1a, b1a, W1b, b1b, W2a, b2a, W2b, b2b):
    src = edge_index[0].astype(jnp.int32).reshape(NROW, KB)
    tgt = edge_index[1].astype(jnp.int32).reshape(NROW, KB)

    g = _gather_rows(x_t, tgt)
    mA, mB = _edge_mlp(g, edge_attr, W1a.T,
                       b1a.reshape(1, H), W1b.T, b1b.reshape(1, H))
    outs = _moments(mA, mB, src)
    sA, sB = outs[0:4], outs[4:8]

    out = _node_mlp(
        x_s, sA, sB, batch_s.reshape(N, 1).astype(jnp.int32),
        u,
        W2a[:, 0:FX].T,
        W2a[:, FX:FX + 1].T,
        W2a[:, 129:273].T,
        W2a[:, 273:417].T,
        W2a[:, 417:561].T,
        W2a[:, 561:705].T,
        W2a[:, 705:833].T,
        b2a.reshape(1, FX),
        W2b.T,
        b2b.reshape(1, FX),
    )
    return out


# speedup vs baseline: 3.4371x; 1.0206x over previous
import functools

import jax
import jax.numpy as jnp
from jax import lax
from jax.experimental import pallas as pl
from jax.experimental.pallas import tpu as pltpu
from jax.experimental.pallas import tpu_sc as plsc

N = 10000
E = 320000
H = 144
FX = 128
NW = 32
KB = 128
NROW = E // KB
RPB = NROW // NW
RPS = N // 16
NCH = H // 16

_mesh = plsc.VectorSubcoreMesh(core_axis_name="c", subcore_axis_name="s")
_sc_params = pltpu.CompilerParams(use_tc_tiling_on_sc=False)


def _leaky(x):
    return jnp.where(x >= 0, x, 0.1 * x)


def _worker_rows(wid):
    base = wid * RPB + jnp.minimum(wid, 4)
    nrow = RPB + (wid < 4).astype(jnp.int32)
    return base, nrow


def _gather_rows(y, tgt2):
    @functools.partial(
        pl.kernel,
        out_type=jax.ShapeDtypeStruct((E, FX), jnp.float32),
        mesh=_mesh,
        compiler_params=_sc_params,
        scratch_types=[
            pltpu.VMEM((RPB + 1, KB), jnp.int32),
            [pltpu.VMEM((KB, FX), jnp.float32) for _ in range(3)],
            pltpu.SemaphoreType.DMA((3,)),
            pltpu.SemaphoreType.DMA((3,)),
        ],
    )
    def k(y_hbm, tgt_hbm, out_hbm, idx_all, rows, g_sem, w_sem):
        wid = lax.axis_index("c") * 16 + lax.axis_index("s")
        base, _ = _worker_rows(wid)
        pltpu.sync_copy(tgt_hbm.at[pl.ds(base, RPB)],
                        idx_all.at[pl.ds(0, RPB)])

        @pl.when(wid < 4)
        def _():
            pltpu.sync_copy(tgt_hbm.at[base + RPB], idx_all.at[RPB])

        def slot(i3, carry):
            for bo in range(3):
                i = 3 * i3 + bo
                b = bo
                bj = (bo + 2) % 3

                @pl.when(i < RPB)
                def _():
                    @pl.when(i >= 3)
                    def _():
                        pltpu.make_async_copy(
                            rows[b],
                            out_hbm.at[pl.ds((base + i - 3) * KB, KB)],
                            w_sem.at[b]).wait()
                    pltpu.async_copy(y_hbm.at[idx_all.at[i]], rows[b],
                                     g_sem.at[b])

                @pl.when((i >= 1) & (i <= RPB))
                def _():
                    j = i - 1
                    pltpu.make_async_copy(y_hbm.at[idx_all.at[j]], rows[bj],
                                          g_sem.at[bj]).wait()
                    pltpu.async_copy(rows[bj],
                                     out_hbm.at[pl.ds((base + j) * KB, KB)],
                                     w_sem.at[bj])
            return carry

        lax.fori_loop(0, (RPB + 1 + 2) // 3, slot, 0)
        for kk in (RPB - 3, RPB - 2, RPB - 1):
            b = kk % 3
            pltpu.make_async_copy(
                rows[b], out_hbm.at[pl.ds((base + kk) * KB, KB)],
                w_sem.at[b]).wait()

        @pl.when(wid < 4)
        def _():
            pltpu.sync_copy(y_hbm.at[idx_all.at[RPB]], rows[0])
            pltpu.sync_copy(rows[0], out_hbm.at[pl.ds((base + RPB) * KB, KB)])

    return k(y, tgt2)


def _edge_mlp(g, ea, w1a_t, b1a, w1b_t, b1b):
    def body(g_ref, ea_ref, w1a_ref, b1a_ref, w1b_ref, b1b_ref,
             oa_ref, ob_ref):
        hp = jax.lax.Precision.HIGHEST
        x = jnp.concatenate([g_ref[...], ea_ref[...]], axis=1)
        a = jnp.dot(x, w1a_ref[...], precision=hp,
                    preferred_element_type=jnp.float32)
        q = _leaky(a + b1a_ref[...])
        res = jnp.dot(q, w1b_ref[...], precision=hp,
                      preferred_element_type=jnp.float32) + b1b_ref[...]
        oa_ref[...] = res[:, 0:FX]
        ob_ref[...] = res[:, 16:H]

    eb = 3200
    return pl.pallas_call(
        body,
        grid=(E // eb,),
        in_specs=[
            pl.BlockSpec((eb, FX), lambda i: (i, 0)),
            pl.BlockSpec((eb, 16), lambda i: (i, 0)),
            pl.BlockSpec((H, H), lambda i: (0, 0)),
            pl.BlockSpec((1, H), lambda i: (0, 0)),
            pl.BlockSpec((H, H), lambda i: (0, 0)),
            pl.BlockSpec((1, H), lambda i: (0, 0)),
        ],
        out_specs=[
            pl.BlockSpec((eb, FX), lambda i: (i, 0)),
            pl.BlockSpec((eb, FX), lambda i: (i, 0)),
        ],
        out_shape=[
            jax.ShapeDtypeStruct((E, FX), jnp.float32),
            jax.ShapeDtypeStruct((E, FX), jnp.float32),
        ],
    )(g, ea, w1a_t, b1a, w1b_t, b1b)


def _moments(mA, mB, src2):
    out_t = tuple(jax.ShapeDtypeStruct((N, FX), jnp.float32)
                  for _ in range(8))

    SPR = NROW // 16
    REM = NROW - 16 * SPR
    HR = NROW // 2
    SPR8 = HR // 16
    REM8 = HR - 16 * SPR8

    @functools.partial(
        pl.kernel,
        out_type=out_t,
        mesh=_mesh,
        compiler_params=_sc_params,
        scratch_types=[
            pltpu.VMEM((SPR + 1, KB), jnp.int32),
            [pltpu.VMEM((KB, 16), jnp.float32) for _ in range(3)],
            [pltpu.VMEM((KB, 64), jnp.float32) for _ in range(3)],
            pltpu.VMEM((KB, 16), jnp.float32),
            pltpu.VMEM((RPS // 5, 64), jnp.float32),
            pltpu.VMEM_SHARED((N, 64), jnp.float32),
            pltpu.VMEM_SHARED((N, 16), jnp.float32),
            pltpu.SemaphoreType.DMA((3,)),
            pltpu.SemaphoreType.DMA((3,)),
            pltpu.SemaphoreType.DMA((3,)),
        ],
    )
    def k(mA_hbm, mB_hbm, src_hbm,
          sA1, sA2, sA3, sA4, sB1, sB2, sB3, sB4,
          idx_all, m_v, pow_v, ones_v, zrow, acc, cacc,
          m_sem, sc_sem, cnt_sem):
        cid = lax.axis_index("c")
        sid = lax.axis_index("s")
        r0 = sid * RPS

        base_f = sid * SPR + jnp.minimum(sid, REM)
        base_8 = cid * HR + sid * SPR8 + jnp.minimum(sid, REM8)

        pltpu.sync_copy(src_hbm.at[pl.ds(base_f, SPR)],
                        idx_all.at[pl.ds(0, SPR)])

        @pl.when(sid < REM)
        def _():
            pltpu.sync_copy(src_hbm.at[base_f + SPR], idx_all.at[SPR])

        zvec = jnp.zeros((16,), jnp.float32)
        lane = lax.iota(jnp.int32, 16)
        onevec = jnp.where(lane == 0, 1.0, 0.0).astype(jnp.float32)

        def zinit(r, carry):
            for c in range(4):
                zrow[r, pl.ds(16 * c, 16)] = zvec
            return carry

        lax.fori_loop(0, RPS // 5, zinit, 0)

        def zero_acc():
            for t in range(5):
                pltpu.sync_copy(zrow,
                                acc.at[pl.ds(r0 + t * (RPS // 5), RPS // 5)])

        def zero_cacc():
            for t in range(5):
                pltpu.sync_copy(zrow.at[:, pl.ds(0, 16)],
                                cacc.at[pl.ds(r0 + t * (RPS // 5), RPS // 5)])

        def oinit(r, carry):
            ones_v[r, pl.ds(0, 16)] = onevec
            return carry

        lax.fori_loop(0, KB, oinit, 0)

        sAs = (sA1, sA2, sA3, sA4)
        sBs = (sB1, sB2, sB3, sB4)

        def compute_pows(b):
            def row(rr, c2):
                m = m_v[b][rr, :]
                m2 = m * m
                pow_v[b][rr, pl.ds(0, 16)] = m
                pow_v[b][rr, pl.ds(16, 16)] = m2
                pow_v[b][rr, pl.ds(32, 16)] = m2 * m
                pow_v[b][rr, pl.ds(48, 16)] = m2 * m2
                return c2

            lax.fori_loop(0, KB, row, 0, unroll=8)

        def run_chunk(src_arr, col, base, nblk, docount):
            def mload_copy(i, b):
                r = base + i
                return pltpu.make_async_copy(
                    src_arr.at[pl.ds(r * KB, KB), pl.ds(col, 16)],
                    m_v[b], m_sem.at[b])

            def scat_copy(i, b):
                return pltpu.make_async_copy(
                    pow_v[b], acc.at[idx_all.at[i]], sc_sem.at[b])

            def slot(i3, carry):
                for bo in range(3):
                    i = 3 * i3 + bo
                    b = bo
                    bj = (bo + 2) % 3

                    @pl.when(i < nblk)
                    def _():
                        mload_copy(i, b).start()

                    @pl.when((i >= 1) & (i <= nblk))
                    def _():
                        j = i - 1
                        mload_copy(j, bj).wait()

                        @pl.when(j >= 3)
                        def _():
                            scat_copy(j - 3, bj).wait()
                            if docount:
                                pltpu.make_async_copy(
                                    ones_v, cacc.at[idx_all.at[j - 3]],
                                    cnt_sem.at[bj]).wait()

                        compute_pows(bj)
                        pltpu.async_copy(pow_v[bj], acc.at[idx_all.at[j]],
                                         sc_sem.at[bj], add=True)
                        if docount:
                            pltpu.async_copy(ones_v, cacc.at[idx_all.at[j]],
                                             cnt_sem.at[bj], add=True)
                return carry

            lax.fori_loop(0, (nblk + 1 + 2) // 3, slot, 0)
            for kk in (nblk - 3, nblk - 2, nblk - 1):
                b = kk % 3
                scat_copy(kk, b).wait()
                if docount:
                    pltpu.make_async_copy(ones_v, cacc.at[idx_all.at[kk]],
                                          cnt_sem.at[b]).wait()

        def tail_block(src_arr, col, r, irow, docount):
            pltpu.sync_copy(src_arr.at[pl.ds(r * KB, KB), pl.ds(col, 16)],
                            m_v[0])
            compute_pows(0)
            pltpu.sync_copy(pow_v[0], acc.at[idx_all.at[irow]], add=True)
            if docount:
                pltpu.sync_copy(ones_v, cacc.at[idx_all.at[irow]], add=True)

        for j in range(NCH):
            src_arr = mA_hbm if j < 8 else mB_hbm
            col = 16 * j if j < 8 else 112

            if j < 8:
                @pl.when(cid == (j % 2))
                def _():
                    zero_acc()
                    if j == 1:
                        zero_cacc()
                    plsc.subcore_barrier()
                    run_chunk(src_arr, col, base_f, SPR, j == 1)

                    @pl.when(sid < REM)
                    def _():
                        tail_block(src_arr, col, base_f + SPR, SPR, j == 1)

                    plsc.subcore_barrier()
                    for q in range(4):
                        pltpu.sync_copy(
                            acc.at[pl.ds(r0, RPS), pl.ds(16 * q, 16)],
                            sAs[q].at[pl.ds(r0, RPS), pl.ds(16 * j, 16)])
                    if j == 1:
                        pltpu.sync_copy(cacc.at[pl.ds(r0, RPS)],
                                        sB1.at[pl.ds(r0, RPS),
                                               pl.ds(16, 16)])
            else:
                pltpu.sync_copy(src_hbm.at[pl.ds(base_8, SPR8)],
                                idx_all.at[pl.ds(0, SPR8)])

                @pl.when(sid < REM8)
                def _():
                    pltpu.sync_copy(src_hbm.at[base_8 + SPR8],
                                    idx_all.at[SPR8])

                zero_acc()
                plsc.subcore_barrier()
                run_chunk(src_arr, col, base_8, SPR8, False)

                @pl.when(sid < REM8)
                def _():
                    tail_block(src_arr, col, base_8 + SPR8, SPR8, False)

                plsc.subcore_barrier()
                dcol = cid * 32
                for q in range(4):
                    pltpu.sync_copy(
                        acc.at[pl.ds(r0, RPS), pl.ds(16 * q, 16)],
                        sBs[q].at[pl.ds(r0, RPS), pl.ds(dcol, 16)])

    return k(mA, mB, src2)


def _node_mlp(x_s, sA, sB, bs, u,
              a_xs, a_cnt, a_m, a_sd, a_sk, a_ku, a_u, b2a, w2b_t, b2b):
    def body(xs_ref, sA1_ref, sA2_ref, sA3_ref, sA4_ref,
             sB1_ref, sB2_ref, sB3_ref, sB4_ref, bs_ref, u_ref,
             axs_ref, acnt_ref, am_ref, asd_ref, ask_ref, aku_ref, au_ref,
             b2a_ref, w2b_ref, b2b_ref, o_ref):
        tail = lambda b_ref: b_ref[:, 0:16] + b_ref[:, 32:48]
        s1v = jnp.concatenate([sA1_ref[...], tail(sB1_ref)], axis=1)
        s2v = jnp.concatenate([sA2_ref[...], tail(sB2_ref)], axis=1)
        s3v = jnp.concatenate([sA3_ref[...], tail(sB3_ref)], axis=1)
        s4v = jnp.concatenate([sA4_ref[...], tail(sB4_ref)], axis=1)
        count = sB1_ref[:, 16:17]
        cnt = jnp.maximum(count, 1.0)
        inv = 1.0 / cnt
        mu = s1v * inv
        m2 = s2v * inv
        mu2 = mu * mu
        var = jnp.maximum(m2 - mu2, 0.0)
        v1 = var + 1e-6
        std = jnp.sqrt(v1)
        c3m = s3v * inv - 3.0 * mu * m2 + 2.0 * mu2 * mu
        c4m = s4v * inv - 4.0 * mu * (s3v * inv) + 6.0 * mu2 * m2 \
            - 3.0 * mu2 * mu2
        skew = c3m / (std * v1)
        kurt = c4m / (v1 * v1)

        dot = lambda a, b: jnp.dot(a, b, precision=jax.lax.Precision.HIGHEST,
                                   preferred_element_type=jnp.float32)
        uproj = dot(u_ref[...], au_ref[...])
        ids = jax.lax.broadcasted_iota(jnp.int32, (bs_ref.shape[0], 64), 1)
        oneh = (bs_ref[...] == ids).astype(jnp.float32)

        h = dot(xs_ref[...], axs_ref[...]) + count * acnt_ref[...] \
            + dot(mu, am_ref[...]) + dot(std, asd_ref[...]) \
            + dot(skew, ask_ref[...]) + dot(kurt, aku_ref[...]) \
            + dot(oneh, uproj) + b2a_ref[...]
        h = _leaky(h)
        o_ref[...] = dot(h, w2b_ref[...]) + b2b_ref[...]

    nb = 1000
    full = lambda r, c: pl.BlockSpec((r, c), lambda i: (0, 0))
    row_blk = lambda c: pl.BlockSpec((nb, c), lambda i: (i, 0))
    return pl.pallas_call(
        body,
        grid=(N // nb,),
        in_specs=[
            row_blk(FX),
            row_blk(FX), row_blk(FX), row_blk(FX), row_blk(FX),
            row_blk(FX), row_blk(FX), row_blk(FX), row_blk(FX),
            pl.BlockSpec((nb, 1), lambda i: (i, 0)),
            full(64, FX),
            full(FX, FX),
            full(1, FX),
            full(H, FX),
            full(H, FX),
            full(H, FX),
            full(H, FX),
            full(FX, FX),
            full(1, FX),
            full(FX, FX),
            full(1, FX),
        ],
        out_specs=pl.BlockSpec((nb, FX), lambda i: (i, 0)),
        out_shape=jax.ShapeDtypeStruct((N, FX), jnp.float32),
    )(x_s, *sA, *sB, bs, u,
      a_xs, a_cnt, a_m, a_sd, a_sk, a_ku, a_u, b2a, w2b_t, b2b)


def kernel(x_s, x_t, edge_index, edge_attr, u, batch_s,
           W1a, b1a, W1b, b1b, W2a, b2a, W2b, b2b):
    src = edge_index[0].astype(jnp.int32).reshape(NROW, KB)
    tgt = edge_index[1].astype(jnp.int32).reshape(NROW, KB)

    g = _gather_rows(x_t, tgt)
    mA, mB = _edge_mlp(g, edge_attr, W1a.T,
                       b1a.reshape(1, H), W1b.T, b1b.reshape(1, H))
    outs = _moments(mA, mB, src)
    sA, sB = outs[0:4], outs[4:8]

    out = _node_mlp(
        x_s, sA, sB, batch_s.reshape(N, 1).astype(jnp.int32),
        u,
        W2a[:, 0:FX].T,
        W2a[:, FX:FX + 1].T,
        W2a[:, 129:273].T,
        W2a[:, 273:417].T,
        W2a[:, 417:561].T,
        W2a[:, 561:705].T,
        W2a[:, 705:833].T,
        b2a.reshape(1, FX),
        W2b.T,
        b2b.reshape(1, FX),
    )
    return out


# speedup vs baseline: 4.6629x; 1.3566x over previous
import functools

import jax
import jax.numpy as jnp
from jax import lax
from jax.experimental import pallas as pl
from jax.experimental.pallas import tpu as pltpu
from jax.experimental.pallas import tpu_sc as plsc

N = 10000
E = 320000
H = 144
FX = 128
NW = 32
KB = 128
NROW = E // KB
RPB = NROW // NW
RPS = N // 16
NCH = H // 16

_mesh = plsc.VectorSubcoreMesh(core_axis_name="c", subcore_axis_name="s")
_sc_params = pltpu.CompilerParams(use_tc_tiling_on_sc=False)


def _leaky(x):
    return jnp.where(x >= 0, x, 0.1 * x)


def _worker_rows(wid):
    base = wid * RPB + jnp.minimum(wid, 4)
    nrow = RPB + (wid < 4).astype(jnp.int32)
    return base, nrow


def _gather_rows(y, tgt2):
    @functools.partial(
        pl.kernel,
        out_type=jax.ShapeDtypeStruct((E, FX), jnp.float32),
        mesh=_mesh,
        compiler_params=_sc_params,
        scratch_types=[
            pltpu.VMEM((RPB + 1, KB), jnp.int32),
            [pltpu.VMEM((KB, FX), jnp.float32) for _ in range(3)],
            pltpu.SemaphoreType.DMA((3,)),
            pltpu.SemaphoreType.DMA((3,)),
        ],
    )
    def k(y_hbm, tgt_hbm, out_hbm, idx_all, rows, g_sem, w_sem):
        wid = lax.axis_index("c") * 16 + lax.axis_index("s")
        base, _ = _worker_rows(wid)
        pltpu.sync_copy(tgt_hbm.at[pl.ds(base, RPB)],
                        idx_all.at[pl.ds(0, RPB)])

        @pl.when(wid < 4)
        def _():
            pltpu.sync_copy(tgt_hbm.at[base + RPB], idx_all.at[RPB])

        def slot(i3, carry):
            for bo in range(3):
                i = 3 * i3 + bo
                b = bo
                bj = (bo + 2) % 3

                @pl.when(i < RPB)
                def _():
                    @pl.when(i >= 3)
                    def _():
                        pltpu.make_async_copy(
                            rows[b],
                            out_hbm.at[pl.ds((base + i - 3) * KB, KB)],
                            w_sem.at[b]).wait()
                    pltpu.async_copy(y_hbm.at[idx_all.at[i]], rows[b],
                                     g_sem.at[b])

                @pl.when((i >= 1) & (i <= RPB))
                def _():
                    j = i - 1
                    pltpu.make_async_copy(y_hbm.at[idx_all.at[j]], rows[bj],
                                          g_sem.at[bj]).wait()
                    pltpu.async_copy(rows[bj],
                                     out_hbm.at[pl.ds((base + j) * KB, KB)],
                                     w_sem.at[bj])
            return carry

        lax.fori_loop(0, (RPB + 1 + 2) // 3, slot, 0)
        for kk in (RPB - 3, RPB - 2, RPB - 1):
            b = kk % 3
            pltpu.make_async_copy(
                rows[b], out_hbm.at[pl.ds((base + kk) * KB, KB)],
                w_sem.at[b]).wait()

        @pl.when(wid < 4)
        def _():
            pltpu.sync_copy(y_hbm.at[idx_all.at[RPB]], rows[0])
            pltpu.sync_copy(rows[0], out_hbm.at[pl.ds((base + RPB) * KB, KB)])

    return k(y, tgt2)


def _split_bf16(w):
    hi = w.astype(jnp.bfloat16)
    lo = (w - hi.astype(jnp.float32)).astype(jnp.bfloat16)
    return hi, lo


def _edge_mlp(g, ea, wah, wal, b1a, wbh, wbl, b1b):
    def body(g_ref, ea_ref, wah_ref, wal_ref, b1a_ref, wbh_ref, wbl_ref,
             b1b_ref, oa_ref, ob_ref):
        d = lambda p, q: jnp.dot(p, q, preferred_element_type=jnp.float32)
        x = jnp.concatenate([g_ref[...], ea_ref[...]], axis=1)
        xh = x.astype(jnp.bfloat16)
        xl = (x - xh.astype(jnp.float32)).astype(jnp.bfloat16)
        a = d(xh, wah_ref[...]) + (d(xh, wal_ref[...])
                                   + d(xl, wah_ref[...])) + b1a_ref[...]
        q = _leaky(a)
        qh = q.astype(jnp.bfloat16)
        ql = (q - qh.astype(jnp.float32)).astype(jnp.bfloat16)
        res = d(qh, wbh_ref[...]) + (d(qh, wbl_ref[...])
                                     + d(ql, wbh_ref[...])) + b1b_ref[...]
        oa_ref[...] = res[:, 0:FX]
        ob_ref[...] = res[:, 16:H]

    eb = 3200
    wspec = pl.BlockSpec((H, H), lambda i: (0, 0))
    return pl.pallas_call(
        body,
        grid=(E // eb,),
        in_specs=[
            pl.BlockSpec((eb, FX), lambda i: (i, 0)),
            pl.BlockSpec((eb, 16), lambda i: (i, 0)),
            wspec, wspec,
            pl.BlockSpec((1, H), lambda i: (0, 0)),
            wspec, wspec,
            pl.BlockSpec((1, H), lambda i: (0, 0)),
        ],
        out_specs=[
            pl.BlockSpec((eb, FX), lambda i: (i, 0)),
            pl.BlockSpec((eb, FX), lambda i: (i, 0)),
        ],
        out_shape=[
            jax.ShapeDtypeStruct((E, FX), jnp.float32),
            jax.ShapeDtypeStruct((E, FX), jnp.float32),
        ],
    )(g, ea, wah, wal, b1a, wbh, wbl, b1b)


def _moments(mA, mB, src2):
    out_t = tuple(jax.ShapeDtypeStruct((N, FX), jnp.float32)
                  for _ in range(8))

    SPR = NROW // 16
    REM = NROW - 16 * SPR
    HR = NROW // 2
    SPR8 = HR // 16
    REM8 = HR - 16 * SPR8

    @functools.partial(
        pl.kernel,
        out_type=out_t,
        mesh=_mesh,
        compiler_params=_sc_params,
        scratch_types=[
            pltpu.VMEM((SPR + 1, KB), jnp.int32),
            [pltpu.VMEM((KB, 16), jnp.float32) for _ in range(3)],
            [pltpu.VMEM((KB, 64), jnp.float32) for _ in range(3)],
            pltpu.VMEM((KB, 16), jnp.float32),
            pltpu.VMEM((RPS // 5, 64), jnp.float32),
            pltpu.VMEM_SHARED((N, 64), jnp.float32),
            pltpu.VMEM_SHARED((N, 16), jnp.float32),
            pltpu.SemaphoreType.DMA((3,)),
            pltpu.SemaphoreType.DMA((3,)),
            pltpu.SemaphoreType.DMA((3,)),
        ],
    )
    def k(mA_hbm, mB_hbm, src_hbm,
          sA1, sA2, sA3, sA4, sB1, sB2, sB3, sB4,
          idx_all, m_v, pow_v, ones_v, zrow, acc, cacc,
          m_sem, sc_sem, cnt_sem):
        cid = lax.axis_index("c")
        sid = lax.axis_index("s")
        r0 = sid * RPS

        base_f = sid * SPR + jnp.minimum(sid, REM)
        base_8 = cid * HR + sid * SPR8 + jnp.minimum(sid, REM8)

        pltpu.sync_copy(src_hbm.at[pl.ds(base_f, SPR)],
                        idx_all.at[pl.ds(0, SPR)])

        @pl.when(sid < REM)
        def _():
            pltpu.sync_copy(src_hbm.at[base_f + SPR], idx_all.at[SPR])

        zvec = jnp.zeros((16,), jnp.float32)
        lane = lax.iota(jnp.int32, 16)
        onevec = jnp.where(lane == 0, 1.0, 0.0).astype(jnp.float32)

        def zinit(r, carry):
            for c in range(4):
                zrow[r, pl.ds(16 * c, 16)] = zvec
            return carry

        lax.fori_loop(0, RPS // 5, zinit, 0)

        def zero_acc():
            for t in range(5):
                pltpu.sync_copy(zrow,
                                acc.at[pl.ds(r0 + t * (RPS // 5), RPS // 5)])

        def zero_cacc():
            for t in range(5):
                pltpu.sync_copy(zrow.at[:, pl.ds(0, 16)],
                                cacc.at[pl.ds(r0 + t * (RPS // 5), RPS // 5)])

        def oinit(r, carry):
            ones_v[r, pl.ds(0, 16)] = onevec
            return carry

        lax.fori_loop(0, KB, oinit, 0)

        sAs = (sA1, sA2, sA3, sA4)
        sBs = (sB1, sB2, sB3, sB4)

        def compute_pows(b):
            def row(rr, c2):
                m = m_v[b][rr, :]
                m2 = m * m
                pow_v[b][rr, pl.ds(0, 16)] = m
                pow_v[b][rr, pl.ds(16, 16)] = m2
                pow_v[b][rr, pl.ds(32, 16)] = m2 * m
                pow_v[b][rr, pl.ds(48, 16)] = m2 * m2
                return c2

            lax.fori_loop(0, KB, row, 0, unroll=8)

        def run_chunk(src_arr, col, base, nblk, docount):
            def mload_copy(i, b):
                r = base + i
                return pltpu.make_async_copy(
                    src_arr.at[pl.ds(r * KB, KB), pl.ds(col, 16)],
                    m_v[b], m_sem.at[b])

            def scat_copy(i, b):
                return pltpu.make_async_copy(
                    pow_v[b], acc.at[idx_all.at[i]], sc_sem.at[b])

            def slot(i3, carry):
                for bo in range(3):
                    i = 3 * i3 + bo
                    b = bo
                    bj = (bo + 2) % 3

                    @pl.when(i < nblk)
                    def _():
                        mload_copy(i, b).start()

                    @pl.when((i >= 1) & (i <= nblk))
                    def _():
                        j = i - 1
                        mload_copy(j, bj).wait()

                        @pl.when(j >= 3)
                        def _():
                            scat_copy(j - 3, bj).wait()
                            if docount:
                                pltpu.make_async_copy(
                                    ones_v, cacc.at[idx_all.at[j - 3]],
                                    cnt_sem.at[bj]).wait()

                        compute_pows(bj)
                        pltpu.async_copy(pow_v[bj], acc.at[idx_all.at[j]],
                                         sc_sem.at[bj], add=True)
                        if docount:
                            pltpu.async_copy(ones_v, cacc.at[idx_all.at[j]],
                                             cnt_sem.at[bj], add=True)
                return carry

            lax.fori_loop(0, (nblk + 1 + 2) // 3, slot, 0)
            for kk in (nblk - 3, nblk - 2, nblk - 1):
                b = kk % 3
                scat_copy(kk, b).wait()
                if docount:
                    pltpu.make_async_copy(ones_v, cacc.at[idx_all.at[kk]],
                                          cnt_sem.at[b]).wait()

        def tail_block(src_arr, col, r, irow, docount):
            pltpu.sync_copy(src_arr.at[pl.ds(r * KB, KB), pl.ds(col, 16)],
                            m_v[0])
            compute_pows(0)
            pltpu.sync_copy(pow_v[0], acc.at[idx_all.at[irow]], add=True)
            if docount:
                pltpu.sync_copy(ones_v, cacc.at[idx_all.at[irow]], add=True)

        for j in range(NCH):
            src_arr = mA_hbm if j < 8 else mB_hbm
            col = 16 * j if j < 8 else 112

            if j < 8:
                @pl.when(cid == (j % 2))
                def _():
                    zero_acc()
                    if j == 1:
                        zero_cacc()
                    plsc.subcore_barrier()
                    run_chunk(src_arr, col, base_f, SPR, j == 1)

                    @pl.when(sid < REM)
                    def _():
                        tail_block(src_arr, col, base_f + SPR, SPR, j == 1)

                    plsc.subcore_barrier()
                    for q in range(4):
                        pltpu.sync_copy(
                            acc.at[pl.ds(r0, RPS), pl.ds(16 * q, 16)],
                            sAs[q].at[pl.ds(r0, RPS), pl.ds(16 * j, 16)])
                    if j == 1:
                        pltpu.sync_copy(cacc.at[pl.ds(r0, RPS)],
                                        sB1.at[pl.ds(r0, RPS),
                                               pl.ds(16, 16)])
            else:
                pltpu.sync_copy(src_hbm.at[pl.ds(base_8, SPR8)],
                                idx_all.at[pl.ds(0, SPR8)])

                @pl.when(sid < REM8)
                def _():
                    pltpu.sync_copy(src_hbm.at[base_8 + SPR8],
                                    idx_all.at[SPR8])

                zero_acc()
                plsc.subcore_barrier()
                run_chunk(src_arr, col, base_8, SPR8, False)

                @pl.when(sid < REM8)
                def _():
                    tail_block(src_arr, col, base_8 + SPR8, SPR8, False)

                plsc.subcore_barrier()
                dcol = cid * 32
                for q in range(4):
                    pltpu.sync_copy(
                        acc.at[pl.ds(r0, RPS), pl.ds(16 * q, 16)],
                        sBs[q].at[pl.ds(r0, RPS), pl.ds(dcol, 16)])

    return k(mA, mB, src2)


def _node_mlp(x_s, sA, sB, bs, u,
              a_xs, a_cnt, a_m, a_sd, a_sk, a_ku, a_u, b2a, w2b_t, b2b):
    def body(xs_ref, sA1_ref, sA2_ref, sA3_ref, sA4_ref,
             sB1_ref, sB2_ref, sB3_ref, sB4_ref, bs_ref, u_ref,
             axs_ref, acnt_ref, am_ref, asd_ref, ask_ref, aku_ref, au_ref,
             b2a_ref, w2b_ref, b2b_ref, o_ref):
        tail = lambda b_ref: b_ref[:, 0:16] + b_ref[:, 32:48]
        s1v = jnp.concatenate([sA1_ref[...], tail(sB1_ref)], axis=1)
        s2v = jnp.concatenate([sA2_ref[...], tail(sB2_ref)], axis=1)
        s3v = jnp.concatenate([sA3_ref[...], tail(sB3_ref)], axis=1)
        s4v = jnp.concatenate([sA4_ref[...], tail(sB4_ref)], axis=1)
        count = sB1_ref[:, 16:17]
        cnt = jnp.maximum(count, 1.0)
        inv = 1.0 / cnt
        mu = s1v * inv
        m2 = s2v * inv
        mu2 = mu * mu
        var = jnp.maximum(m2 - mu2, 0.0)
        v1 = var + 1e-6
        std = jnp.sqrt(v1)
        c3m = s3v * inv - 3.0 * mu * m2 + 2.0 * mu2 * mu
        c4m = s4v * inv - 4.0 * mu * (s3v * inv) + 6.0 * mu2 * m2 \
            - 3.0 * mu2 * mu2
        skew = c3m / (std * v1)
        kurt = c4m / (v1 * v1)

        dot = lambda a, b: jnp.dot(a, b, precision=jax.lax.Precision.HIGHEST,
                                   preferred_element_type=jnp.float32)
        uproj = dot(u_ref[...], au_ref[...])
        ids = jax.lax.broadcasted_iota(jnp.int32, (bs_ref.shape[0], 64), 1)
        oneh = (bs_ref[...] == ids).astype(jnp.float32)

        h = dot(xs_ref[...], axs_ref[...]) + count * acnt_ref[...] \
            + dot(mu, am_ref[...]) + dot(std, asd_ref[...]) \
            + dot(skew, ask_ref[...]) + dot(kurt, aku_ref[...]) \
            + dot(oneh, uproj) + b2a_ref[...]
        h = _leaky(h)
        o_ref[...] = dot(h, w2b_ref[...]) + b2b_ref[...]

    nb = 1000
    full = lambda r, c: pl.BlockSpec((r, c), lambda i: (0, 0))
    row_blk = lambda c: pl.BlockSpec((nb, c), lambda i: (i, 0))
    return pl.pallas_call(
        body,
        grid=(N // nb,),
        in_specs=[
            row_blk(FX),
            row_blk(FX), row_blk(FX), row_blk(FX), row_blk(FX),
            row_blk(FX), row_blk(FX), row_blk(FX), row_blk(FX),
            pl.BlockSpec((nb, 1), lambda i: (i, 0)),
            full(64, FX),
            full(FX, FX),
            full(1, FX),
            full(H, FX),
            full(H, FX),
            full(H, FX),
            full(H, FX),
            full(FX, FX),
            full(1, FX),
            full(FX, FX),
            full(1, FX),
        ],
        out_specs=pl.BlockSpec((nb, FX), lambda i: (i, 0)),
        out_shape=jax.ShapeDtypeStruct((N, FX), jnp.float32),
    )(x_s, *sA, *sB, bs, u,
      a_xs, a_cnt, a_m, a_sd, a_sk, a_ku, a_u, b2a, w2b_t, b2b)


def kernel(x_s, x_t, edge_index, edge_attr, u, batch_s,
           W1a, b1a, W1b, b1b, W2a, b2a, W2b, b2b):
    src = edge_index[0].astype(jnp.int32).reshape(NROW, KB)
    tgt = edge_index[1].astype(jnp.int32).reshape(NROW, KB)

    g = _gather_rows(x_t, tgt)
    wah, wal = _split_bf16(W1a.T)
    wbh, wbl = _split_bf16(W1b.T)
    mA, mB = _edge_mlp(g, edge_attr, wah, wal,
                       b1a.reshape(1, H), wbh, wbl, b1b.reshape(1, H))
    outs = _moments(mA, mB, src)
    sA, sB = outs[0:4], outs[4:8]

    out = _node_mlp(
        x_s, sA, sB, batch_s.reshape(N, 1).astype(jnp.int32),
        u,
        W2a[:, 0:FX].T,
        W2a[:, FX:FX + 1].T,
        W2a[:, 129:273].T,
        W2a[:, 273:417].T,
        W2a[:, 417:561].T,
        W2a[:, 561:705].T,
        W2a[:, 705:833].T,
        b2a.reshape(1, FX),
        W2b.T,
        b2b.reshape(1, FX),
    )
    return out
